# Initial kernel scaffold; baseline (speedup 1.0000x reference)
import jax
import jax.numpy as jnp
from jax.experimental import pallas as pl


def kernel(x, edge_index, batch, W1, b1, W2, b2):
    raise NotImplementedError("write your pallas kernel here")


# speedup vs baseline: 8.4804x; 8.4804x over previous
import functools

import jax
import jax.numpy as jnp
from jax import lax
from jax.experimental import pallas as pl
from jax.experimental.pallas import tpu as pltpu
from jax.experimental.pallas import tpu_sc as plsc

_NSUB = 16
_NCORE = 2


def _sc_agg16(table, src, dst, zeros16):
  N = table.shape[0]
  NP = -(-N // 128) * 128
  E = src.shape[0]
  per_tile = E // (_NCORE * _NSUB)
  K = 40
  nchunk = per_tile // K
  rows_per_sub = NP // _NSUB

  mesh = plsc.VectorSubcoreMesh(core_axis_name="c", subcore_axis_name="s")

  @functools.partial(
      pl.kernel,
      out_type=jax.ShapeDtypeStruct((2 * NP, 16), jnp.float32),
      mesh=mesh,
      scratch_types=[
          pltpu.VMEM((K,), jnp.int32),
          pltpu.VMEM((K,), jnp.int32),
          pltpu.VMEM((K, 16), jnp.float32),
          pltpu.VMEM_SHARED((NP, 16), jnp.float32),
          pltpu.SemaphoreType.DMA,
      ],
      compiler_params=pltpu.CompilerParams(use_tc_tiling_on_sc=False),
  )
  def kern(table_h, src_h, dst_h, zeros_h, out_h, src_v, dst_v, rows_v,
           acc_sh, sem):
    c = lax.axis_index("c")
    s = lax.axis_index("s")
    row0 = s * rows_per_sub
    pltpu.sync_copy(zeros_h.at[pl.ds(row0, rows_per_sub)],
                    acc_sh.at[pl.ds(row0, rows_per_sub)])
    plsc.subcore_barrier()

    base = (c * _NSUB + s) * per_tile

    def body(j, carry):
      off = base + j * K
      pltpu.sync_copy(src_h.at[pl.ds(off, K)], src_v)
      pltpu.sync_copy(dst_h.at[pl.ds(off, K)], dst_v)
      pltpu.async_copy(table_h.at[src_v], rows_v, sem).wait()
      pltpu.sync_copy(rows_v, acc_sh.at[dst_v], add=True)
      return carry

    lax.fori_loop(0, nchunk, body, 0)
    plsc.subcore_barrier()
    pltpu.sync_copy(acc_sh.at[pl.ds(row0, rows_per_sub)],
                    out_h.at[pl.ds(c * NP + row0, rows_per_sub)])

  return kern(table, src, dst, zeros16)


def _sc_agg128(xs2, src, dst, zeros128):
  N = xs2.shape[0] // 2
  NP = -(-N // 128) * 128
  E = src.shape[0]
  per_tile = E // _NSUB
  K = 80
  nchunk = per_tile // K
  rows_per_sub = NP // _NSUB

  mesh = plsc.VectorSubcoreMesh(core_axis_name="c", subcore_axis_name="s")

  @functools.partial(
      pl.kernel,
      out_type=jax.ShapeDtypeStruct((2 * NP, 128), jnp.float32),
      mesh=mesh,
      scratch_types=[
          pltpu.VMEM((K,), jnp.int32),
          pltpu.VMEM((K,), jnp.int32),
          pltpu.VMEM((K, 128), jnp.float32),
          pltpu.VMEM_SHARED((NP, 128), jnp.float32),
          pltpu.SemaphoreType.DMA,
      ],
  )
  def kern(xs_h, src_h, dst_h, zeros_h, out_h, src_v, dst_v, rows_v,
           acc_sh, sem):
    c = lax.axis_index("c")
    s = lax.axis_index("s")
    row0 = s * rows_per_sub
    pltpu.sync_copy(zeros_h.at[pl.ds(row0, rows_per_sub)],
                    acc_sh.at[pl.ds(row0, rows_per_sub)])
    plsc.subcore_barrier()

    base = s * per_tile
    cN = c * N

    def body(j, carry):
      off = base + j * K
      pltpu.sync_copy(src_h.at[pl.ds(off, K)], src_v)
      pltpu.sync_copy(dst_h.at[pl.ds(off, K)], dst_v)
      for i in range(K // 16):
        sl = pl.ds(i * 16, 16)
        src_v[sl] = src_v[sl] + cN
      pltpu.async_copy(xs_h.at[src_v], rows_v, sem).wait()
      pltpu.sync_copy(rows_v, acc_sh.at[dst_v], add=True)
      return carry

    lax.fori_loop(0, nchunk, body, 0)
    plsc.subcore_barrier()
    pltpu.sync_copy(acc_sh.at[pl.ds(row0, rows_per_sub)],
                    out_h.at[pl.ds(c * NP + row0, rows_per_sub)])

  return kern(xs2, src, dst, zeros128)


def _tc_prep(deg2, x):
  N, D = x.shape
  B = 2000
  grid = N // B

  def body(deg_ref, x_ref, xs_ref, dinv_ref):
    deg = deg_ref[0, :, 0:1] + deg_ref[1, :, 0:1] + 1.0
    dv = lax.rsqrt(deg)
    dinv_ref[...] = dv
    xs = x_ref[...] * dv
    xs_ref[0] = xs[:, : D // 2]
    xs_ref[1] = xs[:, D // 2:]

  return pl.pallas_call(
      body,
      grid=(grid,),
      in_specs=[
          pl.BlockSpec((2, B, 16), lambda i: (0, i, 0)),
          pl.BlockSpec((B, D), lambda i: (i, 0)),
      ],
      out_specs=[
          pl.BlockSpec((2, B, D // 2), lambda i: (0, i, 0)),
          pl.BlockSpec((B, 1), lambda i: (i, 0)),
      ],
      out_shape=[
          jax.ShapeDtypeStruct((2, N, D // 2), jnp.float32),
          jax.ShapeDtypeStruct((N, 1), jnp.float32),
      ],
  )(deg2, x)


def _tc_main(agg1, xs, dinv, W1, b1, W2p):
  N = dinv.shape[0]
  D = W1.shape[0]
  B = 1000
  grid = N // B

  def body(agg_ref, xs_ref, dinv_ref, w1_ref, b1_ref, w2_ref, hs_ref):
    dv = dinv_ref[...]
    pre = jnp.concatenate(
        [agg_ref[0] + xs_ref[0], agg_ref[1] + xs_ref[1]], axis=1) * dv
    h = jnp.dot(pre, w1_ref[...], preferred_element_type=jnp.float32)
    h = jnp.maximum(h + b1_ref[...], 0.0)
    hs_ref[...] = jnp.dot(
        h, w2_ref[...], preferred_element_type=jnp.float32) * dv

  return pl.pallas_call(
      body,
      grid=(grid,),
      in_specs=[
          pl.BlockSpec((2, B, D // 2), lambda i: (0, i, 0)),
          pl.BlockSpec((2, B, D // 2), lambda i: (0, i, 0)),
          pl.BlockSpec((B, 1), lambda i: (i, 0)),
          pl.BlockSpec(W1.shape, lambda i: (0, 0)),
          pl.BlockSpec((1, W1.shape[1]), lambda i: (0, 0)),
          pl.BlockSpec(W2p.shape, lambda i: (0, 0)),
      ],
      out_specs=pl.BlockSpec((B, 16), lambda i: (i, 0)),
      out_shape=jax.ShapeDtypeStruct((N, 16), jnp.float32),
  )(agg1, xs, dinv, W1, b1, W2p)


def _tc_final(agg2, hs, dinv, b2, batch, num_graphs):
  N = dinv.shape[0]
  B = 2000
  grid = N // B
  G = num_graphs

  def body(agg_ref, hs_ref, dinv_ref, b2_ref, batch_ref, out_ref, acc):
    i = pl.program_id(0)

    @pl.when(i == 0)
    def _():
      acc[...] = jnp.zeros_like(acc)

    z16 = (agg_ref[0] + agg_ref[1] + hs_ref[...]) * dinv_ref[...]
    z = z16[:, 0:2] + b2_ref[...]
    zc = jnp.concatenate([z, jnp.ones((B, 1), jnp.float32)], axis=1)
    oh = (batch_ref[...] ==
          lax.broadcasted_iota(jnp.int32, (B, G), 1)).astype(jnp.float32)
    acc[...] += lax.dot_general(
        oh, zc, (((0,), (0,)), ((), ())), preferred_element_type=jnp.float32)

    @pl.when(i == grid - 1)
    def _():
      sums = acc[:, 0:2]
      cnt = jnp.maximum(acc[:, 2:3], 1.0)
      pooled = sums / cnt
      m = jnp.max(pooled, axis=1, keepdims=True)
      e = jnp.exp(pooled - m)
      out_ref[...] = (pooled - m) - jnp.log(jnp.sum(e, axis=1, keepdims=True))

  return pl.pallas_call(
      body,
      grid=(grid,),
      in_specs=[
          pl.BlockSpec((2, B, 16), lambda i: (0, i, 0)),
          pl.BlockSpec((B, 16), lambda i: (i, 0)),
          pl.BlockSpec((B, 1), lambda i: (i, 0)),
          pl.BlockSpec((1, 2), lambda i: (0, 0)),
          pl.BlockSpec((B, 1), lambda i: (i, 0)),
      ],
      out_specs=pl.BlockSpec((G, 2), lambda i: (0, 0)),
      out_shape=jax.ShapeDtypeStruct((G, 2), jnp.float32),
      scratch_shapes=[pltpu.VMEM((G, 3), jnp.float32)],
  )(agg2, hs, dinv, b2, batch)


def kernel(x, edge_index, batch, W1, b1, W2, b2):
  N, D = x.shape
  G = 64

  NP = -(-N // 128) * 128
  src = edge_index[0]
  dst = edge_index[1]
  zeros16 = jnp.zeros((NP, 16), jnp.float32)
  ones16 = jnp.ones((N, 16), jnp.float32)
  zeros128 = jnp.zeros((NP, 128), jnp.float32)

  deg2 = _sc_agg16(ones16, src, dst, zeros16).reshape(2, NP, 16)[:, :N]

  xs, dinv = _tc_prep(deg2, x)

  agg1 = _sc_agg128(xs.reshape(2 * N, D // 2), src, dst,
                    zeros128).reshape(2, NP, D // 2)[:, :N]

  W2p = jnp.pad(W2, ((0, 0), (0, 16 - W2.shape[1])))
  hs = _tc_main(agg1, xs, dinv, W1, b1.reshape(1, -1), W2p)

  agg2 = _sc_agg16(hs, src, dst, zeros16).reshape(2, NP, 16)[:, :N]

  return _tc_final(agg2, hs, dinv, b2.reshape(1, -1),
                   batch.reshape(N, 1).astype(jnp.int32), G)


# speedup vs baseline: 22.5974x; 2.6646x over previous
import functools

import jax
import jax.numpy as jnp
from jax import lax
from jax.experimental import pallas as pl
from jax.experimental.pallas import tpu as pltpu
from jax.experimental.pallas import tpu_sc as plsc

_NSUB = 16
_NCORE = 2


def _sc_deg(dst3, zeros16):
  _, CH, K = dst3.shape
  NP = zeros16.shape[0]
  rows_per_sub = NP // _NSUB
  GRP = 20

  mesh = plsc.VectorSubcoreMesh(core_axis_name="c", subcore_axis_name="s")

  @functools.partial(
      pl.kernel,
      out_type=jax.ShapeDtypeStruct((2 * NP, 16), jnp.float32),
      mesh=mesh,
      scratch_types=[
          pltpu.VMEM((CH, K), jnp.int32),
          pltpu.VMEM((K, 16), jnp.float32),
          pltpu.VMEM_SHARED((NP, 16), jnp.float32),
          pltpu.SemaphoreType.DMA,
      ],
      compiler_params=pltpu.CompilerParams(use_tc_tiling_on_sc=False),
  )
  def kern(dst3_h, zeros_h, out_h, dst_v, ones_v, acc_sh, sem):
    c = lax.axis_index("c")
    s = lax.axis_index("s")
    wid = c * _NSUB + s
    row0 = s * rows_per_sub
    pltpu.sync_copy(zeros_h.at[pl.ds(row0, rows_per_sub)],
                    acc_sh.at[pl.ds(row0, rows_per_sub)])
    pltpu.sync_copy(dst3_h.at[wid], dst_v)

    def fill(i, carry):
      ones_v[i, :] = jnp.ones((16,), jnp.float32)
      return carry

    lax.fori_loop(0, K, fill, 0)
    plsc.subcore_barrier()

    def grp_body(g, carry):
      def fire(j, cc):
        pltpu.async_copy(ones_v, acc_sh.at[dst_v.at[g * GRP + j]], sem,
                         add=True)
        return cc

      lax.fori_loop(0, GRP, fire, 0)

      def drain(j, cc):
        pltpu.make_async_copy(ones_v, acc_sh.at[dst_v.at[0]], sem).wait()
        return cc

      lax.fori_loop(0, GRP, drain, 0)
      return carry

    lax.fori_loop(0, CH // GRP, grp_body, 0)
    plsc.subcore_barrier()
    pltpu.sync_copy(acc_sh.at[pl.ds(row0, rows_per_sub)],
                    out_h.at[pl.ds(c * NP + row0, rows_per_sub)])

  return kern(dst3, zeros16)


def _sc_agg16(table, src3, dst3, zeros16):
  _, CH, K = src3.shape
  NP = zeros16.shape[0]
  rows_per_sub = NP // _NSUB
  R = 4

  mesh = plsc.VectorSubcoreMesh(core_axis_name="c", subcore_axis_name="s")

  @functools.partial(
      pl.kernel,
      out_type=jax.ShapeDtypeStruct((2 * NP, 16), jnp.float32),
      mesh=mesh,
      scratch_types=[
          pltpu.VMEM((CH, K), jnp.int32),
          pltpu.VMEM((CH, K), jnp.int32),
          [pltpu.VMEM((K, 16), jnp.float32)] * R,
          [pltpu.SemaphoreType.DMA] * R,
          [pltpu.SemaphoreType.DMA] * R,
          pltpu.VMEM_SHARED((NP, 16), jnp.float32),
      ],
      compiler_params=pltpu.CompilerParams(use_tc_tiling_on_sc=False),
  )
  def kern(table_h, src3_h, dst3_h, zeros_h, out_h, src_v, dst_v, bufs,
           gsems, ssems, acc_sh):
    c = lax.axis_index("c")
    s = lax.axis_index("s")
    wid = c * _NSUB + s
    row0 = s * rows_per_sub
    pltpu.sync_copy(zeros_h.at[pl.ds(row0, rows_per_sub)],
                    acc_sh.at[pl.ds(row0, rows_per_sub)])
    pltpu.sync_copy(src3_h.at[wid], src_v)
    pltpu.sync_copy(dst3_h.at[wid], dst_v)
    plsc.subcore_barrier()

    for b in range(R):
      pltpu.async_copy(table_h.at[src_v.at[b]], bufs[b], gsems[b])

    def grp_body(g, carry):
      for b in range(R):
        ch = g * R + b
        pltpu.make_async_copy(table_h.at[src_v.at[0]], bufs[b],
                              gsems[b]).wait()
        pltpu.async_copy(bufs[b], acc_sh.at[dst_v.at[ch]], ssems[b],
                         add=True)
      for b in range(R):
        pltpu.make_async_copy(bufs[b], acc_sh.at[dst_v.at[0]],
                              ssems[b]).wait()
        pltpu.async_copy(table_h.at[src_v.at[(g + 1) * R + b]], bufs[b],
                         gsems[b])
      return carry

    lax.fori_loop(0, CH // R - 1, grp_body, 0)
    for b in range(R):
      ch = CH - R + b
      pltpu.make_async_copy(table_h.at[src_v.at[0]], bufs[b],
                            gsems[b]).wait()
      pltpu.async_copy(bufs[b], acc_sh.at[dst_v.at[ch]], ssems[b], add=True)
    for b in range(R):
      pltpu.make_async_copy(bufs[b], acc_sh.at[dst_v.at[0]], ssems[b]).wait()

    plsc.subcore_barrier()
    pltpu.sync_copy(acc_sh.at[pl.ds(row0, rows_per_sub)],
                    out_h.at[pl.ds(c * NP + row0, rows_per_sub)])

  return kern(table, src3, dst3, zeros16)


def _sc_agg64(xs4, src3, dst3, zeros64):
  N = xs4.shape[0] // 4
  _, CH, K = src3.shape
  NP = zeros64.shape[0]
  rows_per_sub = NP // _NSUB
  R = 5

  mesh = plsc.VectorSubcoreMesh(core_axis_name="c", subcore_axis_name="s")

  @functools.partial(
      pl.kernel,
      out_type=jax.ShapeDtypeStruct((4 * NP, 64), jnp.float32),
      mesh=mesh,
      scratch_types=[
          pltpu.VMEM((CH, K), jnp.int32),
          pltpu.VMEM((CH, K), jnp.int32),
          [pltpu.VMEM((K, 64), jnp.float32)] * R,
          [pltpu.SemaphoreType.DMA] * R,
          [pltpu.SemaphoreType.DMA] * R,
          pltpu.VMEM_SHARED((NP, 64), jnp.float32),
      ],
      compiler_params=pltpu.CompilerParams(use_tc_tiling_on_sc=False),
  )
  def kern(xs_h, src3_h, dst3_h, zeros_h, out_h, src_v, dst_v, bufs,
           gsems, ssems, acc_sh):
    c = lax.axis_index("c")
    s = lax.axis_index("s")
    row0 = s * rows_per_sub
    pltpu.sync_copy(src3_h.at[s], src_v)
    pltpu.sync_copy(dst3_h.at[s], dst_v)

    for p in range(2):
      delta = c * N if p == 0 else 2 * N
      quarter = 2 * p + c
      pltpu.sync_copy(zeros_h.at[pl.ds(row0, rows_per_sub)],
                      acc_sh.at[pl.ds(row0, rows_per_sub)])
      plsc.subcore_barrier()

      def adjust(ch):
        for i in range(K // 16):
          sl = pl.ds(i * 16, 16)
          src_v[ch, sl] = src_v[ch, sl] + delta

      for b in range(R):
        adjust(b)
        pltpu.async_copy(xs_h.at[src_v.at[b]], bufs[b], gsems[b])

      def grp_body(g, carry):
        for b in range(R):
          ch = g * R + b
          pltpu.make_async_copy(xs_h.at[src_v.at[0]], bufs[b],
                                gsems[b]).wait()
          pltpu.async_copy(bufs[b], acc_sh.at[dst_v.at[ch]], ssems[b],
                           add=True)
        for b in range(R):
          ch_next = (g + 1) * R + b
          pltpu.make_async_copy(bufs[b], acc_sh.at[dst_v.at[0]],
                                ssems[b]).wait()
          adjust(ch_next)
          pltpu.async_copy(xs_h.at[src_v.at[ch_next]], bufs[b], gsems[b])
        return carry

      lax.fori_loop(0, CH // R - 1, grp_body, 0)
      for b in range(R):
        ch = CH - R + b
        pltpu.make_async_copy(xs_h.at[src_v.at[0]], bufs[b],
                              gsems[b]).wait()
        pltpu.async_copy(bufs[b], acc_sh.at[dst_v.at[ch]], ssems[b],
                         add=True)
      for b in range(R):
        pltpu.make_async_copy(bufs[b], acc_sh.at[dst_v.at[0]],
                              ssems[b]).wait()

      plsc.subcore_barrier()
      pltpu.sync_copy(acc_sh.at[pl.ds(row0, rows_per_sub)],
                      out_h.at[pl.ds(quarter * NP + row0, rows_per_sub)])

  return kern(xs4, src3, dst3, zeros64)


def _tc_prep(deg2, x):
  N, D = x.shape
  B = 2000
  grid = N // B

  def body(deg_ref, x_ref, xs_ref, dinv_ref):
    deg = deg_ref[0, :, 0:1] + deg_ref[1, :, 0:1] + 1.0
    dv = lax.rsqrt(deg)
    dinv_ref[...] = dv
    xs = x_ref[...] * dv
    for q in range(4):
      xs_ref[q] = xs[:, q * (D // 4):(q + 1) * (D // 4)]

  return pl.pallas_call(
      body,
      grid=(grid,),
      in_specs=[
          pl.BlockSpec((2, B, 16), lambda i: (0, i, 0)),
          pl.BlockSpec((B, D), lambda i: (i, 0)),
      ],
      out_specs=[
          pl.BlockSpec((4, B, D // 4), lambda i: (0, i, 0)),
          pl.BlockSpec((B, 1), lambda i: (i, 0)),
      ],
      out_shape=[
          jax.ShapeDtypeStruct((4, N, D // 4), jnp.float32),
          jax.ShapeDtypeStruct((N, 1), jnp.float32),
      ],
  )(deg2, x)


def _tc_main(agg1, xs, dinv, W1, b1, W2p):
  N = dinv.shape[0]
  D = W1.shape[0]
  B = 1000
  grid = N // B

  def body(agg_ref, xs_ref, dinv_ref, w1_ref, b1_ref, w2_ref, hs_ref):
    dv = dinv_ref[...]
    pre = jnp.concatenate(
        [agg_ref[q] + xs_ref[q] for q in range(4)], axis=1) * dv
    h = jnp.dot(pre, w1_ref[...], preferred_element_type=jnp.float32)
    h = jnp.maximum(h + b1_ref[...], 0.0)
    hs_ref[...] = jnp.dot(
        h, w2_ref[...], preferred_element_type=jnp.float32) * dv

  return pl.pallas_call(
      body,
      grid=(grid,),
      in_specs=[
          pl.BlockSpec((4, B, D // 4), lambda i: (0, i, 0)),
          pl.BlockSpec((4, B, D // 4), lambda i: (0, i, 0)),
          pl.BlockSpec((B, 1), lambda i: (i, 0)),
          pl.BlockSpec(W1.shape, lambda i: (0, 0)),
          pl.BlockSpec((1, W1.shape[1]), lambda i: (0, 0)),
          pl.BlockSpec(W2p.shape, lambda i: (0, 0)),
      ],
      out_specs=pl.BlockSpec((B, 16), lambda i: (i, 0)),
      out_shape=jax.ShapeDtypeStruct((N, 16), jnp.float32),
  )(agg1, xs, dinv, W1, b1, W2p)


def _tc_final(agg2, hs, dinv, b2, batch, num_graphs):
  N = dinv.shape[0]
  B = 2000
  grid = N // B
  G = num_graphs

  def body(agg_ref, hs_ref, dinv_ref, b2_ref, batch_ref, out_ref, acc):
    i = pl.program_id(0)

    @pl.when(i == 0)
    def _():
      acc[...] = jnp.zeros_like(acc)

    z16 = (agg_ref[0] + agg_ref[1] + hs_ref[...]) * dinv_ref[...]
    z = z16[:, 0:2] + b2_ref[...]
    zc = jnp.concatenate([z, jnp.ones((B, 1), jnp.float32)], axis=1)
    oh = (batch_ref[...] ==
          lax.broadcasted_iota(jnp.int32, (B, G), 1)).astype(jnp.float32)
    acc[...] += lax.dot_general(
        oh, zc, (((0,), (0,)), ((), ())), preferred_element_type=jnp.float32)

    @pl.when(i == grid - 1)
    def _():
      sums = acc[:, 0:2]
      cnt = jnp.maximum(acc[:, 2:3], 1.0)
      pooled = sums / cnt
      m = jnp.max(pooled, axis=1, keepdims=True)
      e = jnp.exp(pooled - m)
      out_ref[...] = (pooled - m) - jnp.log(jnp.sum(e, axis=1, keepdims=True))

  return pl.pallas_call(
      body,
      grid=(grid,),
      in_specs=[
          pl.BlockSpec((2, B, 16), lambda i: (0, i, 0)),
          pl.BlockSpec((B, 16), lambda i: (i, 0)),
          pl.BlockSpec((B, 1), lambda i: (i, 0)),
          pl.BlockSpec((1, 2), lambda i: (0, 0)),
          pl.BlockSpec((B, 1), lambda i: (i, 0)),
      ],
      out_specs=pl.BlockSpec((G, 2), lambda i: (0, 0)),
      out_shape=jax.ShapeDtypeStruct((G, 2), jnp.float32),
      scratch_shapes=[pltpu.VMEM((G, 3), jnp.float32)],
  )(agg2, hs, dinv, b2, batch)


def kernel(x, edge_index, batch, W1, b1, W2, b2):
  N, D = x.shape
  E = edge_index.shape[1]
  G = 64
  NP = -(-N // 128) * 128

  src = edge_index[0]
  dst = edge_index[1]
  src16 = src.reshape(_NCORE * _NSUB, -1, 125)
  dst16 = dst.reshape(_NCORE * _NSUB, -1, 125)
  src128 = src.reshape(_NSUB, -1, 80)
  dst128 = dst.reshape(_NSUB, -1, 80)
  zeros16 = jnp.zeros((NP, 16), jnp.float32)
  zeros64 = jnp.zeros((NP, 64), jnp.float32)

  deg2 = _sc_deg(dst16, zeros16).reshape(2, NP, 16)[:, :N]

  xs, dinv = _tc_prep(deg2, x)

  agg1 = _sc_agg64(xs.reshape(4 * N, D // 4), src128, dst128,
                   zeros64).reshape(4, NP, D // 4)[:, :N]

  W2p = jnp.pad(W2, ((0, 0), (0, 16 - W2.shape[1])))
  hs = _tc_main(agg1, xs, dinv, W1, b1.reshape(1, -1), W2p)

  agg2 = _sc_agg16(hs, src16, dst16, zeros16).reshape(2, NP, 16)[:, :N]

  return _tc_final(agg2, hs, dinv, b2.reshape(1, -1),
                   batch.reshape(N, 1).astype(jnp.int32), G)


# speedup vs baseline: 24.7594x; 1.0957x over previous
import functools

import jax
import jax.numpy as jnp
from jax import lax
from jax.experimental import pallas as pl
from jax.experimental.pallas import tpu as pltpu
from jax.experimental.pallas import tpu_sc as plsc

_NSUB = 16
_NCORE = 2


def _sc_deg(dst3, zeros16):
  _, CH, K = dst3.shape
  NP = zeros16.shape[0]
  rows_per_sub = NP // _NSUB
  GRP = 20

  mesh = plsc.VectorSubcoreMesh(core_axis_name="c", subcore_axis_name="s")

  @functools.partial(
      pl.kernel,
      out_type=jax.ShapeDtypeStruct((NP, 128), jnp.float32),
      mesh=mesh,
      scratch_types=[
          pltpu.VMEM((CH, K), jnp.int32),
          pltpu.VMEM((K, 16), jnp.float32),
          pltpu.VMEM_SHARED((NP, 16), jnp.float32),
          pltpu.SemaphoreType.DMA,
      ],
      compiler_params=pltpu.CompilerParams(use_tc_tiling_on_sc=False),
  )
  def kern(dst3_h, zeros_h, out_h, dst_v, ones_v, acc_sh, sem):
    c = lax.axis_index("c")
    s = lax.axis_index("s")
    wid = c * _NSUB + s
    row0 = s * rows_per_sub
    pltpu.sync_copy(zeros_h.at[pl.ds(row0, rows_per_sub)],
                    acc_sh.at[pl.ds(row0, rows_per_sub)])
    pltpu.sync_copy(dst3_h.at[wid], dst_v)

    def fill(i, carry):
      ones_v[i, :] = jnp.ones((16,), jnp.float32)
      return carry

    lax.fori_loop(0, K, fill, 0)
    plsc.subcore_barrier()

    def grp_body(g, carry):
      def fire(j, cc):
        pltpu.async_copy(ones_v, acc_sh.at[dst_v.at[g * GRP + j]], sem,
                         add=True)
        return cc

      lax.fori_loop(0, GRP, fire, 0)

      def drain(j, cc):
        pltpu.make_async_copy(ones_v, acc_sh.at[dst_v.at[0]], sem).wait()
        return cc

      lax.fori_loop(0, GRP, drain, 0)
      return carry

    lax.fori_loop(0, CH // GRP, grp_body, 0)
    plsc.subcore_barrier()
    pltpu.sync_copy(acc_sh.at[pl.ds(row0, rows_per_sub)],
                    out_h.at[pl.ds(row0, rows_per_sub), pl.ds(c * 16, 16)])

  return kern(dst3, zeros16)


def _sc_agg16(table, src3, dst3, zeros16):
  _, CH, K = src3.shape
  NP = zeros16.shape[0]
  rows_per_sub = NP // _NSUB
  R = 4

  mesh = plsc.VectorSubcoreMesh(core_axis_name="c", subcore_axis_name="s")

  @functools.partial(
      pl.kernel,
      out_type=jax.ShapeDtypeStruct((NP, 128), jnp.float32),
      mesh=mesh,
      scratch_types=[
          pltpu.VMEM((CH, K), jnp.int32),
          pltpu.VMEM((CH, K), jnp.int32),
          [pltpu.VMEM((K, 16), jnp.float32)] * R,
          [pltpu.SemaphoreType.DMA] * R,
          [pltpu.SemaphoreType.DMA] * R,
          pltpu.VMEM_SHARED((NP, 16), jnp.float32),
      ],
      compiler_params=pltpu.CompilerParams(use_tc_tiling_on_sc=False),
  )
  def kern(table_h, src3_h, dst3_h, zeros_h, out_h, src_v, dst_v, bufs,
           gsems, ssems, acc_sh):
    c = lax.axis_index("c")
    s = lax.axis_index("s")
    wid = c * _NSUB + s
    row0 = s * rows_per_sub
    pltpu.sync_copy(zeros_h.at[pl.ds(row0, rows_per_sub)],
                    acc_sh.at[pl.ds(row0, rows_per_sub)])
    pltpu.sync_copy(src3_h.at[wid], src_v)
    pltpu.sync_copy(dst3_h.at[wid], dst_v)
    plsc.subcore_barrier()

    for b in range(R):
      pltpu.async_copy(table_h.at[src_v.at[b]], bufs[b], gsems[b])

    def grp_body(g, carry):
      for b in range(R):
        ch = g * R + b
        pltpu.make_async_copy(table_h.at[src_v.at[0]], bufs[b],
                              gsems[b]).wait()
        pltpu.async_copy(bufs[b], acc_sh.at[dst_v.at[ch]], ssems[b],
                         add=True)
      for b in range(R):
        pltpu.make_async_copy(bufs[b], acc_sh.at[dst_v.at[0]],
                              ssems[b]).wait()
        pltpu.async_copy(table_h.at[src_v.at[(g + 1) * R + b]], bufs[b],
                         gsems[b])
      return carry

    lax.fori_loop(0, CH // R - 1, grp_body, 0)
    for b in range(R):
      ch = CH - R + b
      pltpu.make_async_copy(table_h.at[src_v.at[0]], bufs[b],
                            gsems[b]).wait()
      pltpu.async_copy(bufs[b], acc_sh.at[dst_v.at[ch]], ssems[b], add=True)
    for b in range(R):
      pltpu.make_async_copy(bufs[b], acc_sh.at[dst_v.at[0]], ssems[b]).wait()

    plsc.subcore_barrier()
    pltpu.sync_copy(acc_sh.at[pl.ds(row0, rows_per_sub)],
                    out_h.at[pl.ds(row0, rows_per_sub), pl.ds(c * 16, 16)])

  return kern(table, src3, dst3, zeros16)


def _sc_agg64(xs4, src3, dst3, zeros64):
  N = xs4.shape[0] // 4
  _, CH, K = src3.shape
  NP = zeros64.shape[0]
  rows_per_sub = NP // _NSUB
  R = 5

  mesh = plsc.VectorSubcoreMesh(core_axis_name="c", subcore_axis_name="s")

  @functools.partial(
      pl.kernel,
      out_type=[jax.ShapeDtypeStruct((NP, 128), jnp.float32),
                jax.ShapeDtypeStruct((NP, 128), jnp.float32)],
      mesh=mesh,
      scratch_types=[
          pltpu.VMEM((CH, K), jnp.int32),
          pltpu.VMEM((CH, K), jnp.int32),
          [pltpu.VMEM((K, 64), jnp.float32)] * R,
          [pltpu.SemaphoreType.DMA] * R,
          [pltpu.SemaphoreType.DMA] * R,
          pltpu.VMEM_SHARED((NP, 64), jnp.float32),
      ],
      compiler_params=pltpu.CompilerParams(use_tc_tiling_on_sc=False),
  )
  def kern(xs_h, src3_h, dst3_h, zeros_h, out0_h, out1_h, src_v, dst_v,
           bufs, gsems, ssems, acc_sh):
    c = lax.axis_index("c")
    s = lax.axis_index("s")
    row0 = s * rows_per_sub
    pltpu.sync_copy(src3_h.at[s], src_v)
    pltpu.sync_copy(dst3_h.at[s], dst_v)

    for p in range(2):
      delta = c * N if p == 0 else 2 * N
      pltpu.sync_copy(zeros_h.at[pl.ds(row0, rows_per_sub)],
                      acc_sh.at[pl.ds(row0, rows_per_sub)])
      plsc.subcore_barrier()

      def adjust(ch, d=delta):
        for i in range(K // 16):
          sl = pl.ds(i * 16, 16)
          src_v[ch, sl] = src_v[ch, sl] + d

      for b in range(R):
        adjust(b)
        pltpu.async_copy(xs_h.at[src_v.at[b]], bufs[b], gsems[b])

      def grp_body(g, carry):
        for b in range(R):
          ch = g * R + b
          pltpu.make_async_copy(xs_h.at[src_v.at[0]], bufs[b],
                                gsems[b]).wait()
          pltpu.async_copy(bufs[b], acc_sh.at[dst_v.at[ch]], ssems[b],
                           add=True)
        for b in range(R):
          ch_next = (g + 1) * R + b
          pltpu.make_async_copy(bufs[b], acc_sh.at[dst_v.at[0]],
                                ssems[b]).wait()
          adjust(ch_next)
          pltpu.async_copy(xs_h.at[src_v.at[ch_next]], bufs[b], gsems[b])
        return carry

      lax.fori_loop(0, CH // R - 1, grp_body, 0)
      for b in range(R):
        ch = CH - R + b
        pltpu.make_async_copy(xs_h.at[src_v.at[0]], bufs[b],
                              gsems[b]).wait()
        pltpu.async_copy(bufs[b], acc_sh.at[dst_v.at[ch]], ssems[b],
                         add=True)
      for b in range(R):
        pltpu.make_async_copy(bufs[b], acc_sh.at[dst_v.at[0]],
                              ssems[b]).wait()

      plsc.subcore_barrier()

      @pl.when(c == 0)
      def _():
        pltpu.sync_copy(acc_sh.at[pl.ds(row0, rows_per_sub)],
                        out0_h.at[pl.ds(row0, rows_per_sub),
                                  pl.ds(64 * p, 64)])

      @pl.when(c == 1)
      def _():
        pltpu.sync_copy(acc_sh.at[pl.ds(row0, rows_per_sub)],
                        out1_h.at[pl.ds(row0, rows_per_sub),
                                  pl.ds(64 * p, 64)])

  return kern(xs4, src3, dst3, zeros64)


def _tc_prep(deg, x4):
  _, N, DQ = x4.shape
  B = 2000
  grid_i = N // B

  def body(deg_ref, x_ref, xs_ref, dinv_ref):
    d = deg_ref[:, 0:1] + deg_ref[:, 16:17] + 1.0
    dv = lax.rsqrt(d)
    dinv_ref[...] = dv
    xs_ref[...] = x_ref[0] * dv

  return pl.pallas_call(
      body,
      grid=(grid_i, 4),
      in_specs=[
          pl.BlockSpec((B, 128), lambda i, q: (i, 0)),
          pl.BlockSpec((1, B, DQ), lambda i, q: (q, i, 0)),
      ],
      out_specs=[
          pl.BlockSpec((B, DQ), lambda i, q: (q * (N // B) + i, 0)),
          pl.BlockSpec((B, 1), lambda i, q: (i, 0)),
      ],
      out_shape=[
          jax.ShapeDtypeStruct((4 * N, DQ), jnp.float32),
          jax.ShapeDtypeStruct((N, 1), jnp.float32),
      ],
  )(deg, x4)


def _tc_main(agg0, agg1, xs4, dinv, W1, b1, W2p):
  N = dinv.shape[0]
  D = W1.shape[0]
  DQ = D // 4
  B = 1000
  grid = N // B
  nb = N // B

  def body(a0_ref, a1_ref, x0_ref, x1_ref, x2_ref, x3_ref, dinv_ref,
           w1_ref, b1_ref, w2_ref, hs_ref):
    dv = dinv_ref[...]
    pre = jnp.concatenate(
        [a0_ref[:, 0:DQ] + x0_ref[...],
         a1_ref[:, 0:DQ] + x1_ref[...],
         a0_ref[:, DQ:2 * DQ] + x2_ref[...],
         a1_ref[:, DQ:2 * DQ] + x3_ref[...]], axis=1) * dv
    h = jnp.dot(pre, w1_ref[...], preferred_element_type=jnp.float32)
    h = jnp.maximum(h + b1_ref[...], 0.0)
    hs_ref[...] = jnp.dot(
        h, w2_ref[...], preferred_element_type=jnp.float32) * dv

  return pl.pallas_call(
      body,
      grid=(grid,),
      in_specs=[
          pl.BlockSpec((B, 128), lambda i: (i, 0)),
          pl.BlockSpec((B, 128), lambda i: (i, 0)),
          pl.BlockSpec((B, DQ), lambda i: (0 * nb + i, 0)),
          pl.BlockSpec((B, DQ), lambda i: (1 * nb + i, 0)),
          pl.BlockSpec((B, DQ), lambda i: (2 * nb + i, 0)),
          pl.BlockSpec((B, DQ), lambda i: (3 * nb + i, 0)),
          pl.BlockSpec((B, 1), lambda i: (i, 0)),
          pl.BlockSpec(W1.shape, lambda i: (0, 0)),
          pl.BlockSpec((1, W1.shape[1]), lambda i: (0, 0)),
          pl.BlockSpec(W2p.shape, lambda i: (0, 0)),
      ],
      out_specs=pl.BlockSpec((B, 16), lambda i: (i, 0)),
      out_shape=jax.ShapeDtypeStruct((N, 16), jnp.float32),
  )(agg0, agg1, xs4, xs4, xs4, xs4, dinv, W1, b1, W2p)


def _tc_final(agg2, hs, dinv, b2, batch, num_graphs):
  N = dinv.shape[0]
  B = 2000
  grid = N // B
  G = num_graphs

  def body(agg_ref, hs_ref, dinv_ref, b2_ref, batch_ref, out_ref, acc):
    i = pl.program_id(0)

    @pl.when(i == 0)
    def _():
      acc[...] = jnp.zeros_like(acc)

    z16 = (agg_ref[:, 0:16] + agg_ref[:, 16:32] + hs_ref[...]) * dinv_ref[...]
    z = z16[:, 0:2] + b2_ref[...]
    zc = jnp.concatenate([z, jnp.ones((B, 1), jnp.float32)], axis=1)
    oh = (batch_ref[...] ==
          lax.broadcasted_iota(jnp.int32, (B, G), 1)).astype(jnp.float32)
    acc[...] += lax.dot_general(
        oh, zc, (((0,), (0,)), ((), ())), preferred_element_type=jnp.float32)

    @pl.when(i == grid - 1)
    def _():
      sums = acc[:, 0:2]
      cnt = jnp.maximum(acc[:, 2:3], 1.0)
      pooled = sums / cnt
      m = jnp.max(pooled, axis=1, keepdims=True)
      e = jnp.exp(pooled - m)
      out_ref[...] = (pooled - m) - jnp.log(jnp.sum(e, axis=1, keepdims=True))

  return pl.pallas_call(
      body,
      grid=(grid,),
      in_specs=[
          pl.BlockSpec((B, 128), lambda i: (i, 0)),
          pl.BlockSpec((B, 16), lambda i: (i, 0)),
          pl.BlockSpec((B, 1), lambda i: (i, 0)),
          pl.BlockSpec((1, 2), lambda i: (0, 0)),
          pl.BlockSpec((B, 1), lambda i: (i, 0)),
      ],
      out_specs=pl.BlockSpec((G, 2), lambda i: (0, 0)),
      out_shape=jax.ShapeDtypeStruct((G, 2), jnp.float32),
      scratch_shapes=[pltpu.VMEM((G, 3), jnp.float32)],
  )(agg2, hs, dinv, b2, batch)


def kernel(x, edge_index, batch, W1, b1, W2, b2):
  N, D = x.shape
  G = 64
  NP = -(-N // 128) * 128

  src = edge_index[0]
  dst = edge_index[1]
  src16 = src.reshape(_NCORE * _NSUB, -1, 125)
  dst16 = dst.reshape(_NCORE * _NSUB, -1, 125)
  src128 = src.reshape(_NSUB, -1, 80)
  dst128 = dst.reshape(_NSUB, -1, 80)
  zeros16 = jnp.zeros((NP, 16), jnp.float32)
  zeros64 = jnp.zeros((NP, 64), jnp.float32)

  deg = _sc_deg(dst16, zeros16)

  x4 = jnp.transpose(x.reshape(N, 4, D // 4), (1, 0, 2))
  xs4, dinv = _tc_prep(deg, x4)

  agg0, agg1 = _sc_agg64(xs4, src128, dst128, zeros64)

  W2p = jnp.pad(W2, ((0, 0), (0, 16 - W2.shape[1])))
  hs = _tc_main(agg0, agg1, xs4, dinv, W1, b1.reshape(1, -1), W2p)

  agg2 = _sc_agg16(hs, src16, dst16, zeros16)

  return _tc_final(agg2, hs, dinv, b2.reshape(1, -1),
                   batch.reshape(N, 1).astype(jnp.int32), G)


# speedup vs baseline: 28.8529x; 1.1653x over previous
import functools

import jax
import jax.numpy as jnp
from jax import lax
from jax.experimental import pallas as pl
from jax.experimental.pallas import tpu as pltpu
from jax.experimental.pallas import tpu_sc as plsc

_NSUB = 16
_NCORE = 2


def _sc_deg(dst3, zeros16):
  _, CH, K = dst3.shape
  NP = zeros16.shape[0]
  rows_per_sub = NP // _NSUB
  GRP = 20

  mesh = plsc.VectorSubcoreMesh(core_axis_name="c", subcore_axis_name="s")

  @functools.partial(
      pl.kernel,
      out_type=jax.ShapeDtypeStruct((NP, 128), jnp.float32),
      mesh=mesh,
      scratch_types=[
          pltpu.VMEM((CH, K), jnp.int32),
          pltpu.VMEM((K, 16), jnp.float32),
          pltpu.VMEM_SHARED((NP, 16), jnp.float32),
          pltpu.SemaphoreType.DMA,
      ],
      compiler_params=pltpu.CompilerParams(use_tc_tiling_on_sc=False),
  )
  def kern(dst3_h, zeros_h, out_h, dst_v, ones_v, acc_sh, sem):
    c = lax.axis_index("c")
    s = lax.axis_index("s")
    wid = c * _NSUB + s
    row0 = s * rows_per_sub
    pltpu.sync_copy(zeros_h.at[pl.ds(row0, rows_per_sub)],
                    acc_sh.at[pl.ds(row0, rows_per_sub)])
    pltpu.sync_copy(dst3_h.at[wid], dst_v)

    def fill(i, carry):
      ones_v[i, :] = jnp.ones((16,), jnp.float32)
      return carry

    lax.fori_loop(0, K, fill, 0)
    plsc.subcore_barrier()

    def grp_body(g, carry):
      def fire(j, cc):
        pltpu.async_copy(ones_v, acc_sh.at[dst_v.at[g * GRP + j]], sem,
                         add=True)
        return cc

      lax.fori_loop(0, GRP, fire, 0)

      def drain(j, cc):
        pltpu.make_async_copy(ones_v, acc_sh.at[dst_v.at[0]], sem).wait()
        return cc

      lax.fori_loop(0, GRP, drain, 0)
      return carry

    lax.fori_loop(0, CH // GRP, grp_body, 0)
    plsc.subcore_barrier()
    pltpu.sync_copy(acc_sh.at[pl.ds(row0, rows_per_sub)],
                    out_h.at[pl.ds(row0, rows_per_sub), pl.ds(c * 16, 16)])

  return kern(dst3, zeros16)


def _sc_agg16(table, src3, dst3, zeros16):
  _, CH, K = src3.shape
  NP = zeros16.shape[0]
  rows_per_sub = NP // _NSUB
  R = 4

  mesh = plsc.VectorSubcoreMesh(core_axis_name="c", subcore_axis_name="s")

  @functools.partial(
      pl.kernel,
      out_type=jax.ShapeDtypeStruct((NP, 128), jnp.float32),
      mesh=mesh,
      scratch_types=[
          pltpu.VMEM((CH, K), jnp.int32),
          pltpu.VMEM((CH, K), jnp.int32),
          [pltpu.VMEM((K, 16), jnp.float32)] * R,
          [pltpu.SemaphoreType.DMA] * R,
          [pltpu.SemaphoreType.DMA] * R,
          pltpu.VMEM_SHARED((NP, 16), jnp.float32),
      ],
      compiler_params=pltpu.CompilerParams(use_tc_tiling_on_sc=False),
  )
  def kern(table_h, src3_h, dst3_h, zeros_h, out_h, src_v, dst_v, bufs,
           gsems, ssems, acc_sh):
    c = lax.axis_index("c")
    s = lax.axis_index("s")
    wid = c * _NSUB + s
    row0 = s * rows_per_sub
    pltpu.sync_copy(zeros_h.at[pl.ds(row0, rows_per_sub)],
                    acc_sh.at[pl.ds(row0, rows_per_sub)])
    pltpu.sync_copy(src3_h.at[wid], src_v)
    pltpu.sync_copy(dst3_h.at[wid], dst_v)
    plsc.subcore_barrier()

    for b in range(R):
      pltpu.async_copy(table_h.at[src_v.at[b]], bufs[b], gsems[b])

    def grp_body(g, carry):
      for b in range(R):
        ch = g * R + b
        pltpu.make_async_copy(table_h.at[src_v.at[0]], bufs[b],
                              gsems[b]).wait()
        pltpu.async_copy(bufs[b], acc_sh.at[dst_v.at[ch]], ssems[b],
                         add=True)
      for b in range(R):
        pltpu.make_async_copy(bufs[b], acc_sh.at[dst_v.at[0]],
                              ssems[b]).wait()
        pltpu.async_copy(table_h.at[src_v.at[(g + 1) * R + b]], bufs[b],
                         gsems[b])
      return carry

    lax.fori_loop(0, CH // R - 1, grp_body, 0)
    for b in range(R):
      ch = CH - R + b
      pltpu.make_async_copy(table_h.at[src_v.at[0]], bufs[b],
                            gsems[b]).wait()
      pltpu.async_copy(bufs[b], acc_sh.at[dst_v.at[ch]], ssems[b], add=True)
    for b in range(R):
      pltpu.make_async_copy(bufs[b], acc_sh.at[dst_v.at[0]], ssems[b]).wait()

    plsc.subcore_barrier()
    pltpu.sync_copy(acc_sh.at[pl.ds(row0, rows_per_sub)],
                    out_h.at[pl.ds(row0, rows_per_sub), pl.ds(c * 16, 16)])

  return kern(table, src3, dst3, zeros16)


def _sc_agg64(xs4, src3, dst3, zeros64):
  N = xs4.shape[0] // 4
  _, CH, K = src3.shape
  NP = zeros64.shape[0]
  rows_per_sub = NP // _NSUB
  R = 5

  mesh = plsc.VectorSubcoreMesh(core_axis_name="c", subcore_axis_name="s")

  @functools.partial(
      pl.kernel,
      out_type=[jax.ShapeDtypeStruct((NP, 128), jnp.float32),
                jax.ShapeDtypeStruct((NP, 128), jnp.float32)],
      mesh=mesh,
      scratch_types=[
          pltpu.VMEM((CH, K), jnp.int32),
          pltpu.VMEM((CH, K), jnp.int32),
          [pltpu.VMEM((K, 64), jnp.float32)] * R,
          [pltpu.SemaphoreType.DMA] * R,
          [pltpu.SemaphoreType.DMA] * R,
          pltpu.VMEM_SHARED((NP, 64), jnp.float32),
      ],
      compiler_params=pltpu.CompilerParams(use_tc_tiling_on_sc=False),
  )
  def kern(xs_h, src3_h, dst3_h, zeros_h, out0_h, out1_h, src_v, dst_v,
           bufs, gsems, ssems, acc_sh):
    c = lax.axis_index("c")
    s = lax.axis_index("s")
    row0 = s * rows_per_sub
    pltpu.sync_copy(src3_h.at[s], src_v)
    pltpu.sync_copy(dst3_h.at[s], dst_v)

    for p in range(2):
      pltpu.sync_copy(zeros_h.at[pl.ds(row0, rows_per_sub)],
                      acc_sh.at[pl.ds(row0, rows_per_sub)])
      plsc.subcore_barrier()

      def adjust(ch, _p=p):
        for i in range(K // 16):
          sl = pl.ds(i * 16, 16)
          if _p == 0:
            src_v[ch, sl] = src_v[ch, sl] * 4 + c
          else:
            src_v[ch, sl] = src_v[ch, sl] + 2

      for b in range(R):
        adjust(b)
        pltpu.async_copy(xs_h.at[src_v.at[b]], bufs[b], gsems[b])

      def grp_body(g, carry):
        for b in range(R):
          ch = g * R + b
          pltpu.make_async_copy(xs_h.at[src_v.at[0]], bufs[b],
                                gsems[b]).wait()
          pltpu.async_copy(bufs[b], acc_sh.at[dst_v.at[ch]], ssems[b],
                           add=True)
        for b in range(R):
          ch_next = (g + 1) * R + b
          pltpu.make_async_copy(bufs[b], acc_sh.at[dst_v.at[0]],
                                ssems[b]).wait()
          adjust(ch_next)
          pltpu.async_copy(xs_h.at[src_v.at[ch_next]], bufs[b], gsems[b])
        return carry

      lax.fori_loop(0, CH // R - 1, grp_body, 0)
      for b in range(R):
        ch = CH - R + b
        pltpu.make_async_copy(xs_h.at[src_v.at[0]], bufs[b],
                              gsems[b]).wait()
        pltpu.async_copy(bufs[b], acc_sh.at[dst_v.at[ch]], ssems[b],
                         add=True)
      for b in range(R):
        pltpu.make_async_copy(bufs[b], acc_sh.at[dst_v.at[0]],
                              ssems[b]).wait()

      plsc.subcore_barrier()

      @pl.when(c == 0)
      def _():
        pltpu.sync_copy(acc_sh.at[pl.ds(row0, rows_per_sub)],
                        out0_h.at[pl.ds(row0, rows_per_sub),
                                  pl.ds(64 * p, 64)])

      @pl.when(c == 1)
      def _():
        pltpu.sync_copy(acc_sh.at[pl.ds(row0, rows_per_sub)],
                        out1_h.at[pl.ds(row0, rows_per_sub),
                                  pl.ds(64 * p, 64)])

  return kern(xs4, src3, dst3, zeros64)


def _tc_prep(deg, x):
  N, D = x.shape
  B = 2000
  grid_i = N // B

  def body(deg_ref, x_ref, xs_ref, dinv_ref):
    d = deg_ref[:, 0:1] + deg_ref[:, 16:17] + 1.0
    dv = lax.rsqrt(d)
    dinv_ref[...] = dv
    xs_ref[...] = x_ref[...] * dv

  return pl.pallas_call(
      body,
      grid=(grid_i,),
      in_specs=[
          pl.BlockSpec((B, 128), lambda i: (i, 0)),
          pl.BlockSpec((B, D), lambda i: (i, 0)),
      ],
      out_specs=[
          pl.BlockSpec((B, D), lambda i: (i, 0)),
          pl.BlockSpec((B, 1), lambda i: (i, 0)),
      ],
      out_shape=[
          jax.ShapeDtypeStruct((N, D), jnp.float32),
          jax.ShapeDtypeStruct((N, 1), jnp.float32),
      ],
  )(deg, x)


def _tc_main(agg0, agg1, xs, dinv, W1, b1, W2p):
  N, D = xs.shape
  DQ = D // 4
  B = 1000
  grid = N // B

  def body(a0_ref, a1_ref, xs_ref, dinv_ref, w1_ref, b1_ref, w2_ref,
           hs_ref):
    dv = dinv_ref[...]
    pre = jnp.concatenate(
        [a0_ref[:, 0:DQ] + xs_ref[:, 0:DQ],
         a1_ref[:, 0:DQ] + xs_ref[:, DQ:2 * DQ],
         a0_ref[:, DQ:2 * DQ] + xs_ref[:, 2 * DQ:3 * DQ],
         a1_ref[:, DQ:2 * DQ] + xs_ref[:, 3 * DQ:4 * DQ]], axis=1) * dv
    h = jnp.dot(pre, w1_ref[...], preferred_element_type=jnp.float32)
    h = jnp.maximum(h + b1_ref[...], 0.0)
    hs_ref[...] = jnp.dot(
        h, w2_ref[...], preferred_element_type=jnp.float32) * dv

  return pl.pallas_call(
      body,
      grid=(grid,),
      in_specs=[
          pl.BlockSpec((B, 128), lambda i: (i, 0)),
          pl.BlockSpec((B, 128), lambda i: (i, 0)),
          pl.BlockSpec((B, D), lambda i: (i, 0)),
          pl.BlockSpec((B, 1), lambda i: (i, 0)),
          pl.BlockSpec(W1.shape, lambda i: (0, 0)),
          pl.BlockSpec((1, W1.shape[1]), lambda i: (0, 0)),
          pl.BlockSpec(W2p.shape, lambda i: (0, 0)),
      ],
      out_specs=pl.BlockSpec((B, 16), lambda i: (i, 0)),
      out_shape=jax.ShapeDtypeStruct((N, 16), jnp.float32),
  )(agg0, agg1, xs, dinv, W1, b1, W2p)


def _tc_final(agg2, hs, dinv, b2, batch, num_graphs):
  N = dinv.shape[0]
  B = 2000
  grid = N // B
  G = num_graphs

  def body(agg_ref, hs_ref, dinv_ref, b2_ref, batch_ref, out_ref, acc):
    i = pl.program_id(0)

    @pl.when(i == 0)
    def _():
      acc[...] = jnp.zeros_like(acc)

    z16 = (agg_ref[:, 0:16] + agg_ref[:, 16:32] + hs_ref[...]) * dinv_ref[...]
    z = z16[:, 0:2] + b2_ref[...]
    zc = jnp.concatenate([z, jnp.ones((B, 1), jnp.float32)], axis=1)
    oh = (batch_ref[...] ==
          lax.broadcasted_iota(jnp.int32, (B, G), 1)).astype(jnp.float32)
    acc[...] += lax.dot_general(
        oh, zc, (((0,), (0,)), ((), ())), preferred_element_type=jnp.float32)

    @pl.when(i == grid - 1)
    def _():
      sums = acc[:, 0:2]
      cnt = jnp.maximum(acc[:, 2:3], 1.0)
      pooled = sums / cnt
      m = jnp.max(pooled, axis=1, keepdims=True)
      e = jnp.exp(pooled - m)
      out_ref[...] = (pooled - m) - jnp.log(jnp.sum(e, axis=1, keepdims=True))

  return pl.pallas_call(
      body,
      grid=(grid,),
      in_specs=[
          pl.BlockSpec((B, 128), lambda i: (i, 0)),
          pl.BlockSpec((B, 16), lambda i: (i, 0)),
          pl.BlockSpec((B, 1), lambda i: (i, 0)),
          pl.BlockSpec((1, 2), lambda i: (0, 0)),
          pl.BlockSpec((B, 1), lambda i: (i, 0)),
      ],
      out_specs=pl.BlockSpec((G, 2), lambda i: (0, 0)),
      out_shape=jax.ShapeDtypeStruct((G, 2), jnp.float32),
      scratch_shapes=[pltpu.VMEM((G, 3), jnp.float32)],
  )(agg2, hs, dinv, b2, batch)


def kernel(x, edge_index, batch, W1, b1, W2, b2):
  N, D = x.shape
  G = 64
  NP = -(-N // 128) * 128

  src = edge_index[0]
  dst = edge_index[1]
  src16 = src.reshape(_NCORE * _NSUB, -1, 125)
  dst16 = dst.reshape(_NCORE * _NSUB, -1, 125)
  src128 = src.reshape(_NSUB, -1, 80)
  dst128 = dst.reshape(_NSUB, -1, 80)
  zeros16 = jnp.zeros((NP, 16), jnp.float32)
  zeros64 = jnp.zeros((NP, 64), jnp.float32)

  deg = _sc_deg(dst16, zeros16)

  xs, dinv = _tc_prep(deg, x)
  xs4 = xs.reshape(4 * N, D // 4)

  agg0, agg1 = _sc_agg64(xs4, src128, dst128, zeros64)

  W2p = jnp.pad(W2, ((0, 0), (0, 16 - W2.shape[1])))
  hs = _tc_main(agg0, agg1, xs, dinv, W1, b1.reshape(1, -1), W2p)

  agg2 = _sc_agg16(hs, src16, dst16, zeros16)

  return _tc_final(agg2, hs, dinv, b2.reshape(1, -1),
                   batch.reshape(N, 1).astype(jnp.int32), G)


# speedup vs baseline: 33.5634x; 1.1633x over previous
import functools

import jax
import jax.numpy as jnp
from jax import lax
from jax.experimental import pallas as pl
from jax.experimental.pallas import tpu as pltpu
from jax.experimental.pallas import tpu_sc as plsc

_NSUB = 16
_NCORE = 2


def _sc_deg(dst3, zeros16):
  _, CH, K = dst3.shape
  NP = zeros16.shape[0]
  rows_per_sub = NP // _NSUB
  GRP = 20

  mesh = plsc.VectorSubcoreMesh(core_axis_name="c", subcore_axis_name="s")

  @functools.partial(
      pl.kernel,
      out_type=jax.ShapeDtypeStruct((NP, 128), jnp.float32),
      mesh=mesh,
      scratch_types=[
          pltpu.VMEM((CH, K), jnp.int32),
          pltpu.VMEM((K, 16), jnp.float32),
          pltpu.VMEM_SHARED((NP, 16), jnp.float32),
          pltpu.SemaphoreType.DMA,
      ],
      compiler_params=pltpu.CompilerParams(use_tc_tiling_on_sc=False),
  )
  def kern(dst3_h, zeros_h, out_h, dst_v, ones_v, acc_sh, sem):
    c = lax.axis_index("c")
    s = lax.axis_index("s")
    wid = c * _NSUB + s
    row0 = s * rows_per_sub
    pltpu.sync_copy(zeros_h.at[pl.ds(row0, rows_per_sub)],
                    acc_sh.at[pl.ds(row0, rows_per_sub)])
    pltpu.sync_copy(dst3_h.at[wid], dst_v)

    def fill(i, carry):
      ones_v[i, :] = jnp.ones((16,), jnp.float32)
      return carry

    lax.fori_loop(0, K, fill, 0)
    plsc.subcore_barrier()

    def grp_body(g, carry):
      def fire(j, cc):
        pltpu.async_copy(ones_v, acc_sh.at[dst_v.at[g * GRP + j]], sem,
                         add=True)
        return cc

      lax.fori_loop(0, GRP, fire, 0)

      def drain(j, cc):
        pltpu.make_async_copy(ones_v, acc_sh.at[dst_v.at[0]], sem).wait()
        return cc

      lax.fori_loop(0, GRP, drain, 0)
      return carry

    lax.fori_loop(0, CH // GRP, grp_body, 0)
    plsc.subcore_barrier()
    pltpu.sync_copy(acc_sh.at[pl.ds(row0, rows_per_sub)],
                    out_h.at[pl.ds(row0, rows_per_sub), pl.ds(c * 16, 16)])

  return kern(dst3, zeros16)


def _sc_agg16(table, src3, dst3, zeros16):
  _, CH, K = src3.shape
  NP = zeros16.shape[0]
  rows_per_sub = NP // _NSUB
  R = 4

  mesh = plsc.VectorSubcoreMesh(core_axis_name="c", subcore_axis_name="s")

  @functools.partial(
      pl.kernel,
      out_type=jax.ShapeDtypeStruct((NP, 128), jnp.float32),
      mesh=mesh,
      scratch_types=[
          pltpu.VMEM((CH, K), jnp.int32),
          pltpu.VMEM((CH, K), jnp.int32),
          [pltpu.VMEM((K, 16), jnp.float32)] * R,
          [pltpu.SemaphoreType.DMA] * R,
          [pltpu.SemaphoreType.DMA] * R,
          pltpu.VMEM_SHARED((NP, 16), jnp.float32),
      ],
      compiler_params=pltpu.CompilerParams(use_tc_tiling_on_sc=False),
  )
  def kern(table_h, src3_h, dst3_h, zeros_h, out_h, src_v, dst_v, bufs,
           gsems, ssems, acc_sh):
    c = lax.axis_index("c")
    s = lax.axis_index("s")
    wid = c * _NSUB + s
    row0 = s * rows_per_sub
    pltpu.sync_copy(zeros_h.at[pl.ds(row0, rows_per_sub)],
                    acc_sh.at[pl.ds(row0, rows_per_sub)])
    pltpu.sync_copy(src3_h.at[wid], src_v)
    pltpu.sync_copy(dst3_h.at[wid], dst_v)
    plsc.subcore_barrier()

    for b in range(R):
      pltpu.async_copy(table_h.at[src_v.at[b]], bufs[b], gsems[b])

    def grp_body(g, carry):
      for b in range(R):
        ch = g * R + b
        pltpu.make_async_copy(table_h.at[src_v.at[0]], bufs[b],
                              gsems[b]).wait()
        pltpu.async_copy(bufs[b], acc_sh.at[dst_v.at[ch]], ssems[b],
                         add=True)
      for b in range(R):
        pltpu.make_async_copy(bufs[b], acc_sh.at[dst_v.at[0]],
                              ssems[b]).wait()
        pltpu.async_copy(table_h.at[src_v.at[(g + 1) * R + b]], bufs[b],
                         gsems[b])
      return carry

    lax.fori_loop(0, CH // R - 1, grp_body, 0)
    for b in range(R):
      ch = CH - R + b
      pltpu.make_async_copy(table_h.at[src_v.at[0]], bufs[b],
                            gsems[b]).wait()
      pltpu.async_copy(bufs[b], acc_sh.at[dst_v.at[ch]], ssems[b], add=True)
    for b in range(R):
      pltpu.make_async_copy(bufs[b], acc_sh.at[dst_v.at[0]], ssems[b]).wait()

    plsc.subcore_barrier()
    pltpu.sync_copy(acc_sh.at[pl.ds(row0, rows_per_sub)],
                    out_h.at[pl.ds(row0, rows_per_sub), pl.ds(c * 16, 16)])

  return kern(table, src3, dst3, zeros16)


def _sc_agg128(xscat, src3, dst3, zerosb):
  N = xscat.shape[0] // 2
  _, CH, K = src3.shape
  NPB = zerosb.shape[0]
  rows_per_sub = NPB // _NSUB
  R = 5

  mesh = plsc.VectorSubcoreMesh(core_axis_name="c", subcore_axis_name="s")

  @functools.partial(
      pl.kernel,
      out_type=[jax.ShapeDtypeStruct((NPB, 128), jnp.bfloat16),
                jax.ShapeDtypeStruct((NPB, 128), jnp.bfloat16)],
      mesh=mesh,
      scratch_types=[
          pltpu.VMEM((CH, K), jnp.int32),
          pltpu.VMEM((CH, K), jnp.int32),
          [pltpu.VMEM((K, 128), jnp.bfloat16)] * R,
          [pltpu.SemaphoreType.DMA] * R,
          [pltpu.SemaphoreType.DMA] * R,
          pltpu.VMEM_SHARED((NPB, 128), jnp.bfloat16),
      ],
      compiler_params=pltpu.CompilerParams(use_tc_tiling_on_sc=False),
  )
  def kern(xs_h, src3_h, dst3_h, zeros_h, out0_h, out1_h, src_v, dst_v,
           bufs, gsems, ssems, acc_sh):
    c = lax.axis_index("c")
    s = lax.axis_index("s")
    row0 = s * rows_per_sub
    cN = c * N
    pltpu.sync_copy(zeros_h.at[pl.ds(row0, rows_per_sub)],
                    acc_sh.at[pl.ds(row0, rows_per_sub)])
    pltpu.sync_copy(src3_h.at[s], src_v)
    pltpu.sync_copy(dst3_h.at[s], dst_v)
    plsc.subcore_barrier()

    def adjust(ch):
      for i in range(K // 16):
        sl = pl.ds(i * 16, 16)
        src_v[ch, sl] = src_v[ch, sl] + cN

    for b in range(R):
      adjust(b)
      pltpu.async_copy(xs_h.at[src_v.at[b]], bufs[b], gsems[b])

    def grp_body(g, carry):
      for b in range(R):
        ch = g * R + b
        pltpu.make_async_copy(xs_h.at[src_v.at[0]], bufs[b],
                              gsems[b]).wait()
        pltpu.async_copy(bufs[b], acc_sh.at[dst_v.at[ch]], ssems[b],
                         add=True)
      for b in range(R):
        ch_next = (g + 1) * R + b
        pltpu.make_async_copy(bufs[b], acc_sh.at[dst_v.at[0]],
                              ssems[b]).wait()
        adjust(ch_next)
        pltpu.async_copy(xs_h.at[src_v.at[ch_next]], bufs[b], gsems[b])
      return carry

    lax.fori_loop(0, CH // R - 1, grp_body, 0)
    for b in range(R):
      ch = CH - R + b
      pltpu.make_async_copy(xs_h.at[src_v.at[0]], bufs[b], gsems[b]).wait()
      pltpu.async_copy(bufs[b], acc_sh.at[dst_v.at[ch]], ssems[b], add=True)
    for b in range(R):
      pltpu.make_async_copy(bufs[b], acc_sh.at[dst_v.at[0]], ssems[b]).wait()

    plsc.subcore_barrier()

    @pl.when(c == 0)
    def _():
      pltpu.sync_copy(acc_sh.at[pl.ds(row0, rows_per_sub)],
                      out0_h.at[pl.ds(row0, rows_per_sub)])

    @pl.when(c == 1)
    def _():
      pltpu.sync_copy(acc_sh.at[pl.ds(row0, rows_per_sub)],
                      out1_h.at[pl.ds(row0, rows_per_sub)])

  return kern(xscat, src3, dst3, zerosb)


def _tc_prep(deg, x):
  N, D = x.shape
  B = 2000
  grid_i = N // B

  def body(deg_ref, x_ref, xs_ref, dinv_ref):
    d = deg_ref[:, 0:1] + deg_ref[:, 16:17] + 1.0
    dv = lax.rsqrt(d)
    dinv_ref[...] = dv
    xs_ref[...] = (x_ref[...] * dv).astype(jnp.bfloat16)

  return pl.pallas_call(
      body,
      grid=(grid_i, 2),
      in_specs=[
          pl.BlockSpec((B, 128), lambda i, h: (i, 0)),
          pl.BlockSpec((B, 128), lambda i, h: (i, h)),
      ],
      out_specs=[
          pl.BlockSpec((B, 128), lambda i, h: (h * (N // B) + i, 0)),
          pl.BlockSpec((B, 1), lambda i, h: (i, 0)),
      ],
      out_shape=[
          jax.ShapeDtypeStruct((2 * N, 128), jnp.bfloat16),
          jax.ShapeDtypeStruct((N, 1), jnp.float32),
      ],
  )(deg, x)


def _tc_main(agg0, agg1, xscat, dinv, W1, b1, W2p):
  N = dinv.shape[0]
  B = 1000
  grid = N // B
  nb = N // B

  def body(a0_ref, a1_ref, xsl_ref, xsr_ref, dinv_ref, w1_ref, b1_ref,
           w2_ref, hs_ref):
    dv = dinv_ref[...]
    f32 = jnp.float32
    pre = jnp.concatenate(
        [(a0_ref[...] + xsl_ref[...]).astype(f32),
         (a1_ref[...] + xsr_ref[...]).astype(f32)], axis=1) * dv
    h = jnp.dot(pre, w1_ref[...], preferred_element_type=jnp.float32)
    h = jnp.maximum(h + b1_ref[...], 0.0)
    hs_ref[...] = jnp.dot(
        h, w2_ref[...], preferred_element_type=jnp.float32) * dv

  return pl.pallas_call(
      body,
      grid=(grid,),
      in_specs=[
          pl.BlockSpec((B, 128), lambda i: (i, 0)),
          pl.BlockSpec((B, 128), lambda i: (i, 0)),
          pl.BlockSpec((B, 128), lambda i: (i, 0)),
          pl.BlockSpec((B, 128), lambda i: (nb + i, 0)),
          pl.BlockSpec((B, 1), lambda i: (i, 0)),
          pl.BlockSpec(W1.shape, lambda i: (0, 0)),
          pl.BlockSpec((1, W1.shape[1]), lambda i: (0, 0)),
          pl.BlockSpec(W2p.shape, lambda i: (0, 0)),
      ],
      out_specs=pl.BlockSpec((B, 16), lambda i: (i, 0)),
      out_shape=jax.ShapeDtypeStruct((N, 16), jnp.float32),
  )(agg0, agg1, xscat, xscat, dinv, W1, b1, W2p)


def _tc_final(agg2, hs, dinv, b2, batch, num_graphs):
  N = dinv.shape[0]
  B = 2000
  grid = N // B
  G = num_graphs

  def body(agg_ref, hs_ref, dinv_ref, b2_ref, batch_ref, out_ref, acc):
    i = pl.program_id(0)

    @pl.when(i == 0)
    def _():
      acc[...] = jnp.zeros_like(acc)

    z16 = (agg_ref[:, 0:16] + agg_ref[:, 16:32] + hs_ref[...]) * dinv_ref[...]
    z = z16[:, 0:2] + b2_ref[...]
    zc = jnp.concatenate([z, jnp.ones((B, 1), jnp.float32)], axis=1)
    oh = (batch_ref[...] ==
          lax.broadcasted_iota(jnp.int32, (B, G), 1)).astype(jnp.float32)
    acc[...] += lax.dot_general(
        oh, zc, (((0,), (0,)), ((), ())), preferred_element_type=jnp.float32)

    @pl.when(i == grid - 1)
    def _():
      sums = acc[:, 0:2]
      cnt = jnp.maximum(acc[:, 2:3], 1.0)
      pooled = sums / cnt
      m = jnp.max(pooled, axis=1, keepdims=True)
      e = jnp.exp(pooled - m)
      out_ref[...] = (pooled - m) - jnp.log(jnp.sum(e, axis=1, keepdims=True))

  return pl.pallas_call(
      body,
      grid=(grid,),
      in_specs=[
          pl.BlockSpec((B, 128), lambda i: (i, 0)),
          pl.BlockSpec((B, 16), lambda i: (i, 0)),
          pl.BlockSpec((B, 1), lambda i: (i, 0)),
          pl.BlockSpec((1, 2), lambda i: (0, 0)),
          pl.BlockSpec((B, 1), lambda i: (i, 0)),
      ],
      out_specs=pl.BlockSpec((G, 2), lambda i: (0, 0)),
      out_shape=jax.ShapeDtypeStruct((G, 2), jnp.float32),
      scratch_shapes=[pltpu.VMEM((G, 3), jnp.float32)],
  )(agg2, hs, dinv, b2, batch)


def kernel(x, edge_index, batch, W1, b1, W2, b2):
  N, D = x.shape
  G = 64
  NP = -(-N // 128) * 128

  src = edge_index[0]
  dst = edge_index[1]
  src16 = src.reshape(_NCORE * _NSUB, -1, 125)
  dst16 = dst.reshape(_NCORE * _NSUB, -1, 125)
  src128 = src.reshape(_NSUB, -1, 80)
  dst128 = dst.reshape(_NSUB, -1, 80)
  NPB = -(-N // 256) * 256
  zeros16 = jnp.zeros((NP, 16), jnp.float32)
  zerosb = jnp.zeros((NPB, 128), jnp.bfloat16)

  deg = _sc_deg(dst16, zeros16)

  xscat, dinv = _tc_prep(deg, x)

  agg0, agg1 = _sc_agg128(xscat, src128, dst128, zerosb)

  W2p = jnp.pad(W2, ((0, 0), (0, 16 - W2.shape[1])))
  hs = _tc_main(agg0, agg1, xscat, dinv, W1, b1.reshape(1, -1), W2p)

  agg2 = _sc_agg16(hs, src16, dst16, zeros16)

  return _tc_final(agg2, hs, dinv, b2.reshape(1, -1),
                   batch.reshape(N, 1).astype(jnp.int32), G)


# speedup vs baseline: 33.6435x; 1.0024x over previous
import functools

import jax
import jax.numpy as jnp
from jax import lax
from jax.experimental import pallas as pl
from jax.experimental.pallas import tpu as pltpu
from jax.experimental.pallas import tpu_sc as plsc

_NSUB = 16
_NCORE = 2


def _sc_deg(dst3, zeros16):
  _, CH, K = dst3.shape
  NP = zeros16.shape[0]
  rows_per_sub = NP // _NSUB
  GRP = 20

  mesh = plsc.VectorSubcoreMesh(core_axis_name="c", subcore_axis_name="s")

  @functools.partial(
      pl.kernel,
      out_type=jax.ShapeDtypeStruct((NP, 128), jnp.float32),
      mesh=mesh,
      scratch_types=[
          pltpu.VMEM((CH, K), jnp.int32),
          pltpu.VMEM((K, 16), jnp.float32),
          pltpu.VMEM_SHARED((NP, 16), jnp.float32),
          pltpu.SemaphoreType.DMA,
      ],
      compiler_params=pltpu.CompilerParams(use_tc_tiling_on_sc=False),
  )
  def kern(dst3_h, zeros_h, out_h, dst_v, ones_v, acc_sh, sem):
    c = lax.axis_index("c")
    s = lax.axis_index("s")
    wid = c * _NSUB + s
    row0 = s * rows_per_sub
    pltpu.sync_copy(zeros_h.at[pl.ds(row0, rows_per_sub)],
                    acc_sh.at[pl.ds(row0, rows_per_sub)])
    pltpu.sync_copy(dst3_h.at[wid], dst_v)

    def fill(i, carry):
      ones_v[i, :] = jnp.ones((16,), jnp.float32)
      return carry

    lax.fori_loop(0, K, fill, 0)
    plsc.subcore_barrier()

    def grp_body(g, carry):
      def fire(j, cc):
        pltpu.async_copy(ones_v, acc_sh.at[dst_v.at[g * GRP + j]], sem,
                         add=True)
        return cc

      lax.fori_loop(0, GRP, fire, 0)

      def drain(j, cc):
        pltpu.make_async_copy(ones_v, acc_sh.at[dst_v.at[0]], sem).wait()
        return cc

      lax.fori_loop(0, GRP, drain, 0)
      return carry

    lax.fori_loop(0, CH // GRP, grp_body, 0)
    plsc.subcore_barrier()
    pltpu.sync_copy(acc_sh.at[pl.ds(row0, rows_per_sub)],
                    out_h.at[pl.ds(row0, rows_per_sub), pl.ds(c * 16, 16)])

  return kern(dst3, zeros16)


def _sc_agg16(table, src3, dst3, zeros16):
  _, CH, K = src3.shape
  NP = zeros16.shape[0]
  rows_per_sub = NP // _NSUB
  R = 4

  mesh = plsc.VectorSubcoreMesh(core_axis_name="c", subcore_axis_name="s")

  @functools.partial(
      pl.kernel,
      out_type=jax.ShapeDtypeStruct((NP, 128), jnp.float32),
      mesh=mesh,
      scratch_types=[
          pltpu.VMEM((CH, K), jnp.int32),
          pltpu.VMEM((CH, K), jnp.int32),
          [pltpu.VMEM((K, 16), jnp.float32)] * R,
          [pltpu.SemaphoreType.DMA] * R,
          [pltpu.SemaphoreType.DMA] * R,
          pltpu.VMEM_SHARED((NP, 16), jnp.float32),
      ],
      compiler_params=pltpu.CompilerParams(use_tc_tiling_on_sc=False),
  )
  def kern(table_h, src3_h, dst3_h, zeros_h, out_h, src_v, dst_v, bufs,
           gsems, ssems, acc_sh):
    c = lax.axis_index("c")
    s = lax.axis_index("s")
    wid = c * _NSUB + s
    row0 = s * rows_per_sub
    pltpu.sync_copy(zeros_h.at[pl.ds(row0, rows_per_sub)],
                    acc_sh.at[pl.ds(row0, rows_per_sub)])
    pltpu.sync_copy(src3_h.at[wid], src_v)
    pltpu.sync_copy(dst3_h.at[wid], dst_v)
    plsc.subcore_barrier()

    for b in range(R):
      pltpu.async_copy(table_h.at[src_v.at[b]], bufs[b], gsems[b])

    def grp_body(g, carry):
      for b in range(R):
        ch = g * R + b
        pltpu.make_async_copy(table_h.at[src_v.at[0]], bufs[b],
                              gsems[b]).wait()
        pltpu.async_copy(bufs[b], acc_sh.at[dst_v.at[ch]], ssems[b],
                         add=True)
      for b in range(R):
        pltpu.make_async_copy(bufs[b], acc_sh.at[dst_v.at[0]],
                              ssems[b]).wait()
        pltpu.async_copy(table_h.at[src_v.at[(g + 1) * R + b]], bufs[b],
                         gsems[b])
      return carry

    lax.fori_loop(0, CH // R - 1, grp_body, 0)
    for b in range(R):
      ch = CH - R + b
      pltpu.make_async_copy(table_h.at[src_v.at[0]], bufs[b],
                            gsems[b]).wait()
      pltpu.async_copy(bufs[b], acc_sh.at[dst_v.at[ch]], ssems[b], add=True)
    for b in range(R):
      pltpu.make_async_copy(bufs[b], acc_sh.at[dst_v.at[0]], ssems[b]).wait()

    plsc.subcore_barrier()
    pltpu.sync_copy(acc_sh.at[pl.ds(row0, rows_per_sub)],
                    out_h.at[pl.ds(row0, rows_per_sub), pl.ds(c * 16, 16)])

  return kern(table, src3, dst3, zeros16)


def _sc_agg128(xscat, src3, dst3, zerosb):
  N = xscat.shape[0] // 2
  _, CH, K = src3.shape
  NPB = zerosb.shape[0]
  rows_per_sub = NPB // _NSUB
  R = 5

  mesh = plsc.VectorSubcoreMesh(core_axis_name="c", subcore_axis_name="s")

  @functools.partial(
      pl.kernel,
      out_type=[jax.ShapeDtypeStruct((NPB, 128), jnp.bfloat16),
                jax.ShapeDtypeStruct((NPB, 128), jnp.bfloat16)],
      mesh=mesh,
      scratch_types=[
          pltpu.VMEM((CH, K), jnp.int32),
          pltpu.VMEM((CH, K), jnp.int32),
          [pltpu.VMEM((K, 128), jnp.bfloat16)] * R,
          [pltpu.SemaphoreType.DMA] * R,
          [pltpu.SemaphoreType.DMA] * R,
          pltpu.VMEM_SHARED((NPB, 128), jnp.bfloat16),
      ],
      compiler_params=pltpu.CompilerParams(use_tc_tiling_on_sc=False),
  )
  def kern(xs_h, src3_h, dst3_h, zeros_h, out0_h, out1_h, src_v, dst_v,
           bufs, gsems, ssems, acc_sh):
    c = lax.axis_index("c")
    s = lax.axis_index("s")
    row0 = s * rows_per_sub
    cN = c * N
    pltpu.sync_copy(zeros_h.at[pl.ds(row0, rows_per_sub)],
                    acc_sh.at[pl.ds(row0, rows_per_sub)])
    pltpu.sync_copy(src3_h.at[s], src_v)
    pltpu.sync_copy(dst3_h.at[s], dst_v)
    plsc.subcore_barrier()

    def adjust(ch):
      for i in range(K // 16):
        sl = pl.ds(i * 16, 16)
        src_v[ch, sl] = src_v[ch, sl] + cN

    for b in range(R):
      adjust(b)
      pltpu.async_copy(xs_h.at[src_v.at[b]], bufs[b], gsems[b])

    def grp_body(g, carry):
      for b in range(R):
        ch = g * R + b
        pltpu.make_async_copy(xs_h.at[src_v.at[0]], bufs[b],
                              gsems[b]).wait()
        pltpu.async_copy(bufs[b], acc_sh.at[dst_v.at[ch]], ssems[b],
                         add=True)
      for b in range(R):
        ch_next = (g + 1) * R + b
        pltpu.make_async_copy(bufs[b], acc_sh.at[dst_v.at[0]],
                              ssems[b]).wait()
        adjust(ch_next)
        pltpu.async_copy(xs_h.at[src_v.at[ch_next]], bufs[b], gsems[b])
      return carry

    lax.fori_loop(0, CH // R - 1, grp_body, 0)
    for b in range(R):
      ch = CH - R + b
      pltpu.make_async_copy(xs_h.at[src_v.at[0]], bufs[b], gsems[b]).wait()
      pltpu.async_copy(bufs[b], acc_sh.at[dst_v.at[ch]], ssems[b], add=True)
    for b in range(R):
      pltpu.make_async_copy(bufs[b], acc_sh.at[dst_v.at[0]], ssems[b]).wait()

    plsc.subcore_barrier()

    @pl.when(c == 0)
    def _():
      pltpu.sync_copy(acc_sh.at[pl.ds(row0, rows_per_sub)],
                      out0_h.at[pl.ds(row0, rows_per_sub)])

    @pl.when(c == 1)
    def _():
      pltpu.sync_copy(acc_sh.at[pl.ds(row0, rows_per_sub)],
                      out1_h.at[pl.ds(row0, rows_per_sub)])

  return kern(xscat, src3, dst3, zerosb)


def _tc_prep(deg, x):
  N, D = x.shape
  B = 2000
  grid_i = N // B

  def body(deg_ref, x_ref, xs_ref, dinv_ref):
    d = deg_ref[:, 0:1] + deg_ref[:, 16:17] + 1.0
    dv = lax.rsqrt(d)
    dinv_ref[...] = dv
    xs_ref[...] = (x_ref[...] * dv).astype(jnp.bfloat16)

  return pl.pallas_call(
      body,
      grid=(grid_i, 2),
      in_specs=[
          pl.BlockSpec((B, 128), lambda i, h: (i, 0)),
          pl.BlockSpec((B, 128), lambda i, h: (i, h)),
      ],
      out_specs=[
          pl.BlockSpec((B, 128), lambda i, h: (h * (N // B) + i, 0)),
          pl.BlockSpec((B, 1), lambda i, h: (i, 0)),
      ],
      out_shape=[
          jax.ShapeDtypeStruct((2 * N, 128), jnp.bfloat16),
          jax.ShapeDtypeStruct((N, 1), jnp.float32),
      ],
  )(deg, x)


def _tc_main(agg0, agg1, xscat, dinv, W1, b1, W2p):
  N = dinv.shape[0]
  B = 1000
  grid = N // B
  nb = N // B

  def body(a0_ref, a1_ref, xsl_ref, xsr_ref, dinv_ref, w1_ref, b1_ref,
           w2_ref, hs_ref):
    dv = dinv_ref[...]
    f32 = jnp.float32
    pre = (jnp.concatenate(
        [a0_ref[...] + xsl_ref[...],
         a1_ref[...] + xsr_ref[...]], axis=1).astype(f32)
           * dv).astype(jnp.bfloat16)
    h = jnp.dot(pre, w1_ref[...].astype(jnp.bfloat16),
                preferred_element_type=jnp.float32)
    h = jnp.maximum(h + b1_ref[...], 0.0)
    hs_ref[...] = jnp.dot(
        h, w2_ref[...], preferred_element_type=jnp.float32) * dv

  return pl.pallas_call(
      body,
      grid=(grid,),
      in_specs=[
          pl.BlockSpec((B, 128), lambda i: (i, 0)),
          pl.BlockSpec((B, 128), lambda i: (i, 0)),
          pl.BlockSpec((B, 128), lambda i: (i, 0)),
          pl.BlockSpec((B, 128), lambda i: (nb + i, 0)),
          pl.BlockSpec((B, 1), lambda i: (i, 0)),
          pl.BlockSpec(W1.shape, lambda i: (0, 0)),
          pl.BlockSpec((1, W1.shape[1]), lambda i: (0, 0)),
          pl.BlockSpec(W2p.shape, lambda i: (0, 0)),
      ],
      out_specs=pl.BlockSpec((B, 16), lambda i: (i, 0)),
      out_shape=jax.ShapeDtypeStruct((N, 16), jnp.float32),
  )(agg0, agg1, xscat, xscat, dinv, W1, b1, W2p)


def _tc_final(agg2, hs, dinv, b2, batch, num_graphs):
  N = dinv.shape[0]
  B = 2000
  grid = N // B
  G = num_graphs

  def body(agg_ref, hs_ref, dinv_ref, b2_ref, batch_ref, out_ref, acc):
    i = pl.program_id(0)

    @pl.when(i == 0)
    def _():
      acc[...] = jnp.zeros_like(acc)

    z16 = (agg_ref[:, 0:16] + agg_ref[:, 16:32] + hs_ref[...]) * dinv_ref[...]
    z = z16[:, 0:2] + b2_ref[...]
    zc = jnp.concatenate([z, jnp.ones((B, 1), jnp.float32)], axis=1)
    oh = (batch_ref[...] ==
          lax.broadcasted_iota(jnp.int32, (B, G), 1)).astype(jnp.float32)
    acc[...] += lax.dot_general(
        oh, zc, (((0,), (0,)), ((), ())), preferred_element_type=jnp.float32)

    @pl.when(i == grid - 1)
    def _():
      sums = acc[:, 0:2]
      cnt = jnp.maximum(acc[:, 2:3], 1.0)
      pooled = sums / cnt
      m = jnp.max(pooled, axis=1, keepdims=True)
      e = jnp.exp(pooled - m)
      out_ref[...] = (pooled - m) - jnp.log(jnp.sum(e, axis=1, keepdims=True))

  return pl.pallas_call(
      body,
      grid=(grid,),
      in_specs=[
          pl.BlockSpec((B, 128), lambda i: (i, 0)),
          pl.BlockSpec((B, 16), lambda i: (i, 0)),
          pl.BlockSpec((B, 1), lambda i: (i, 0)),
          pl.BlockSpec((1, 2), lambda i: (0, 0)),
          pl.BlockSpec((B, 1), lambda i: (i, 0)),
      ],
      out_specs=pl.BlockSpec((G, 2), lambda i: (0, 0)),
      out_shape=jax.ShapeDtypeStruct((G, 2), jnp.float32),
      scratch_shapes=[pltpu.VMEM((G, 3), jnp.float32)],
  )(agg2, hs, dinv, b2, batch)


def kernel(x, edge_index, batch, W1, b1, W2, b2):
  N, D = x.shape
  G = 64
  NP = -(-N // 128) * 128

  src = edge_index[0]
  dst = edge_index[1]
  src16 = src.reshape(_NCORE * _NSUB, -1, 125)
  dst16 = dst.reshape(_NCORE * _NSUB, -1, 125)
  src128 = src.reshape(_NSUB, -1, 80)
  dst128 = dst.reshape(_NSUB, -1, 80)
  NPB = -(-N // 256) * 256
  zeros16 = jnp.zeros((NP, 16), jnp.float32)
  zerosb = jnp.zeros((NPB, 128), jnp.bfloat16)

  deg = _sc_deg(dst16, zeros16)

  xscat, dinv = _tc_prep(deg, x)

  agg0, agg1 = _sc_agg128(xscat, src128, dst128, zerosb)

  W2p = jnp.pad(W2, ((0, 0), (0, 16 - W2.shape[1])))
  hs = _tc_main(agg0, agg1, xscat, dinv, W1, b1.reshape(1, -1), W2p)

  agg2 = _sc_agg16(hs, src16, dst16, zeros16)

  return _tc_final(agg2, hs, dinv, b2.reshape(1, -1),
                   batch.reshape(N, 1).astype(jnp.int32), G)
